# Initial kernel scaffold; baseline (speedup 1.0000x reference)
import jax
import jax.numpy as jnp
from jax.experimental import pallas as pl


def kernel(z, embedding_weight):
    raise NotImplementedError("write your pallas kernel here")


# speedup vs baseline: 1.1817x; 1.1817x over previous
import jax
import jax.numpy as jnp
from jax.experimental import pallas as pl
from jax.experimental.pallas import tpu as pltpu
from jax.experimental.pallas import tpu_sc as plsc

_N_E = 8192
_E_DIM = 32
_BETA = 0.25
_ROW_BLOCK = 256
_NBLK = 2
_BW = _N_E // _NBLK
_GATHER_WINDOW = 128
_GATHER_ROW = 128


def _dist_argmin_body(zn_ref, z_ref, et_ref, en_ref, idx_ref, dsel_ref):
    zv = z_ref[...].astype(jnp.bfloat16).astype(jnp.float32)
    et = et_ref[...]
    prod = jax.lax.dot_general(
        zv, et, (((1,), (0,)), ((), ())),
        preferred_element_type=jnp.float32,
    )
    d = (zn_ref[...] + en_ref[...]) - 2.0 * prod
    iota = jax.lax.broadcasted_iota(jnp.int32, (_ROW_BLOCK, _BW), 1)
    v = gi = dv = None
    for q in range(_NBLK):
        dq = d[:, q * _BW:(q + 1) * _BW]
        mq = jnp.min(dq, axis=1)
        jq = jnp.min(jnp.where(dq == mq[:, None], iota, _BW), axis=1) + q * _BW
        if v is None:
            v, gi, dv = mq, jq, mq
        else:
            take = (mq < v) | ((mq == v) & (jq < gi))
            v = jnp.where(take, mq, v)
            gi = jnp.where(take, jq, gi)
            dv = jnp.where(take, mq, dv)
        v = v.astype(jnp.bfloat16).astype(jnp.float32)
    idx_ref[0, 0, :] = gi
    dsel_ref[0, 0, :] = dv


def _dist_argmin(z_flat, zn, e_t, en):
    n_rows = z_flat.shape[0]
    n_blocks = n_rows // _ROW_BLOCK
    idx, dsel = pl.pallas_call(
        _dist_argmin_body,
        grid=(n_blocks,),
        in_specs=[
            pl.BlockSpec((_ROW_BLOCK, 1), lambda i: (i, 0)),
            pl.BlockSpec((_ROW_BLOCK, _E_DIM), lambda i: (i, 0)),
            pl.BlockSpec((_E_DIM, _N_E), lambda i: (0, 0)),
            pl.BlockSpec((1, _N_E), lambda i: (0, 0)),
        ],
        out_specs=[
            pl.BlockSpec((1, 1, _ROW_BLOCK), lambda i: (i, 0, 0)),
            pl.BlockSpec((1, 1, _ROW_BLOCK), lambda i: (i, 0, 0)),
        ],
        out_shape=[
            jax.ShapeDtypeStruct((n_blocks, 1, _ROW_BLOCK), jnp.int32),
            jax.ShapeDtypeStruct((n_blocks, 1, _ROW_BLOCK), jnp.float32),
        ],
    )(zn, z_flat, e_t, en)
    return idx.reshape(n_rows), dsel.reshape(n_rows)


def _sc_gather_rows(table_padded, indices_flat):
    n = indices_flat.shape[0]
    idx2d = indices_flat.reshape(1, n)
    mesh = plsc.VectorSubcoreMesh(
        core_axis_name="core", subcore_axis_name="subcore"
    )

    @pl.kernel(
        out_type=jax.ShapeDtypeStruct((n, _GATHER_ROW), table_padded.dtype),
        mesh=mesh,
    )
    def gather_kernel(tbl_hbm, i_hbm, o_hbm):
        def body(i_vmem, o_vmem):
            pltpu.sync_copy(tbl_hbm.at[i_vmem.at[0]], o_vmem)

        pltpu.emit_pipeline(
            body,
            grid=(n // _GATHER_WINDOW,),
            in_specs=[
                pl.BlockSpec((1, _GATHER_WINDOW), lambda i: (0, i))
            ],
            out_specs=[
                pl.BlockSpec((_GATHER_WINDOW, _GATHER_ROW), lambda i: (i, 0))
            ],
            core_axis_name="subcore",
            dimension_semantics=(pltpu.PARALLEL,),
        )(i_hbm, o_hbm)

    return gather_kernel(table_padded, idx2d)


def kernel(z, embedding_weight):
    b, t, _ = z.shape
    z_flat = z.reshape(-1, _E_DIM)
    zn = jnp.sum(z_flat**2, axis=1, keepdims=True)
    en = jnp.sum(embedding_weight**2, axis=1)[None, :]
    e_t = embedding_weight.T

    idx_flat, dsel = _dist_argmin(z_flat, zn, e_t, en)

    table_padded = jnp.pad(embedding_weight, ((0, 0), (0, _GATHER_ROW - _E_DIM)))
    zq_flat = _sc_gather_rows(table_padded, idx_flat)[:, :_E_DIM]
    z_q = zq_flat.reshape(z.shape)

    n_elems = z_flat.shape[0] * _E_DIM
    m = jnp.sum(dsel) / n_elems
    loss = _BETA * m + m

    z_q = z + jax.lax.stop_gradient(z_q - z)
    min_encoding_indices = idx_flat.reshape(b, t)
    v = jnp.var(min_encoding_indices.astype(jnp.float32), ddof=1)
    return (z_q, loss, min_encoding_indices, v)


# speedup vs baseline: 1.2050x; 1.0197x over previous
import jax
import jax.numpy as jnp
from jax.experimental import pallas as pl
from jax.experimental.pallas import tpu as pltpu
from jax.experimental.pallas import tpu_sc as plsc

_N_E = 8192
_E_DIM = 32
_BETA = 0.25
_ROW_BLOCK = 512
_NBLK = 2
_BW = _N_E // _NBLK
_GATHER_WINDOW = 128
_GATHER_ROW = 128


def _dist_argmin_body(zn_ref, z_ref, et_ref, en_ref, idx_ref, dsel_ref):
    zv = z_ref[...].astype(jnp.bfloat16).astype(jnp.float32)
    et = et_ref[...]
    prod = jax.lax.dot_general(
        zv, et, (((1,), (0,)), ((), ())),
        preferred_element_type=jnp.float32,
    )
    d = (zn_ref[...] + en_ref[...]) - 2.0 * prod
    iota = jax.lax.broadcasted_iota(jnp.int32, (_ROW_BLOCK, _BW), 1)
    v = gi = dv = None
    for q in range(_NBLK):
        dq = d[:, q * _BW:(q + 1) * _BW]
        mq = jnp.min(dq, axis=1)
        jq = jnp.min(jnp.where(dq == mq[:, None], iota, _BW), axis=1) + q * _BW
        if v is None:
            v, gi, dv = mq, jq, mq
        else:
            take = (mq < v) | ((mq == v) & (jq < gi))
            v = jnp.where(take, mq, v)
            gi = jnp.where(take, jq, gi)
            dv = jnp.where(take, mq, dv)
        v = v.astype(jnp.bfloat16).astype(jnp.float32)
    idx_ref[0, 0, :] = gi
    dsel_ref[0, 0, :] = dv


def _dist_argmin(z_flat, zn, e_t, en):
    n_rows = z_flat.shape[0]
    n_blocks = n_rows // _ROW_BLOCK
    idx, dsel = pl.pallas_call(
        _dist_argmin_body,
        grid=(n_blocks,),
        in_specs=[
            pl.BlockSpec((_ROW_BLOCK, 1), lambda i: (i, 0)),
            pl.BlockSpec((_ROW_BLOCK, _E_DIM), lambda i: (i, 0)),
            pl.BlockSpec((_E_DIM, _N_E), lambda i: (0, 0)),
            pl.BlockSpec((1, _N_E), lambda i: (0, 0)),
        ],
        out_specs=[
            pl.BlockSpec((1, 1, _ROW_BLOCK), lambda i: (i, 0, 0)),
            pl.BlockSpec((1, 1, _ROW_BLOCK), lambda i: (i, 0, 0)),
        ],
        out_shape=[
            jax.ShapeDtypeStruct((n_blocks, 1, _ROW_BLOCK), jnp.int32),
            jax.ShapeDtypeStruct((n_blocks, 1, _ROW_BLOCK), jnp.float32),
        ],
    )(zn, z_flat, e_t, en)
    return idx.reshape(n_rows), dsel.reshape(n_rows)


def _sc_gather_rows(table_padded, indices_flat):
    n = indices_flat.shape[0]
    idx2d = indices_flat.reshape(1, n)
    mesh = plsc.VectorSubcoreMesh(
        core_axis_name="core", subcore_axis_name="subcore"
    )

    @pl.kernel(
        out_type=jax.ShapeDtypeStruct((n, _GATHER_ROW), table_padded.dtype),
        mesh=mesh,
    )
    def gather_kernel(tbl_hbm, i_hbm, o_hbm):
        def body(i_vmem, o_vmem):
            pltpu.sync_copy(tbl_hbm.at[i_vmem.at[0]], o_vmem)

        pltpu.emit_pipeline(
            body,
            grid=(n // _GATHER_WINDOW,),
            in_specs=[
                pl.BlockSpec((1, _GATHER_WINDOW), lambda i: (0, i))
            ],
            out_specs=[
                pl.BlockSpec((_GATHER_WINDOW, _GATHER_ROW), lambda i: (i, 0))
            ],
            core_axis_name="subcore",
            dimension_semantics=(pltpu.PARALLEL,),
        )(i_hbm, o_hbm)

    return gather_kernel(table_padded, idx2d)


def kernel(z, embedding_weight):
    b, t, _ = z.shape
    z_flat = z.reshape(-1, _E_DIM)
    zn = jnp.sum(z_flat**2, axis=1, keepdims=True)
    en = jnp.sum(embedding_weight**2, axis=1)[None, :]
    e_t = embedding_weight.T

    idx_flat, dsel = _dist_argmin(z_flat, zn, e_t, en)

    table_padded = jnp.pad(embedding_weight, ((0, 0), (0, _GATHER_ROW - _E_DIM)))
    zq_flat = _sc_gather_rows(table_padded, idx_flat)[:, :_E_DIM]
    z_q = zq_flat.reshape(z.shape)

    n_elems = z_flat.shape[0] * _E_DIM
    m = jnp.sum(dsel) / n_elems
    loss = _BETA * m + m

    z_q = z + jax.lax.stop_gradient(z_q - z)
    min_encoding_indices = idx_flat.reshape(b, t)
    v = jnp.var(min_encoding_indices.astype(jnp.float32), ddof=1)
    return (z_q, loss, min_encoding_indices, v)
